# Initial kernel scaffold; baseline (speedup 1.0000x reference)
import jax
import jax.numpy as jnp
from jax.experimental import pallas as pl


def kernel(x, edge_index, W1, b1, W2, b2, Wl, bl):
    raise NotImplementedError("write your pallas kernel here")


# speedup vs baseline: 9.7302x; 9.7302x over previous
import functools

import jax
import jax.numpy as jnp
from jax import lax
from jax.experimental import pallas as pl
from jax.experimental.pallas import tpu as pltpu
from jax.experimental.pallas import tpu_sc as plsc

N = 10000
D = 128
OUT = 40
NP = 10240
RPT = NP // 16
E = 320000
CH = 128
KCH = 79
EPT = CH * KCH
EP = EPT * 32

_MESH = dict(core_axis_name="c", subcore_axis_name="s")


DW = 128


@functools.partial(
    pl.kernel,
    out_type=jax.ShapeDtypeStruct((2, NP, DW), jnp.float32),
    mesh=plsc.VectorSubcoreMesh(**_MESH),
    scratch_types=[
        pltpu.VMEM((CH,), jnp.int32),
        pltpu.VMEM((CH, DW), jnp.float32),
        pltpu.VMEM_SHARED((NP, DW), jnp.float32),
    ],
)
def _deg_kernel(dst_hbm, e1_hbm, z16_hbm, out_hbm, idx_v, e1_v, acc_sh):
    c = lax.axis_index("c")
    s = lax.axis_index("s")
    r0 = s * RPT
    pltpu.sync_copy(z16_hbm, acc_sh.at[pl.ds(r0, RPT)])
    pltpu.sync_copy(e1_hbm, e1_v)
    plsc.subcore_barrier()
    base0 = (c * 16 + s) * EPT

    def body(k, carry):
        pltpu.sync_copy(dst_hbm.at[pl.ds(base0 + k * CH, CH)], idx_v)
        pltpu.sync_copy(e1_v, acc_sh.at[idx_v], add=True)
        return carry

    lax.fori_loop(0, KCH, body, 0)
    plsc.subcore_barrier()
    pltpu.sync_copy(acc_sh.at[pl.ds(r0, RPT)], out_hbm.at[c, pl.ds(r0, RPT)])


@functools.partial(
    pl.kernel,
    out_type=jax.ShapeDtypeStruct((2, NP, D), jnp.float32),
    mesh=plsc.VectorSubcoreMesh(**_MESH),
    scratch_types=[
        pltpu.VMEM((CH,), jnp.int32),
        pltpu.VMEM((CH,), jnp.int32),
        pltpu.VMEM((CH, D), jnp.float32),
        pltpu.VMEM_SHARED((NP, D), jnp.float32),
        pltpu.SemaphoreType.DMA,
    ],
)
def _mp_kernel(z_hbm, src_hbm, dst_hbm, zrow_hbm, out_hbm,
               sidx_v, didx_v, rows_v, acc_sh, sem):
    c = lax.axis_index("c")
    s = lax.axis_index("s")
    r0 = s * RPT
    pltpu.sync_copy(zrow_hbm, acc_sh.at[pl.ds(r0, RPT)])
    plsc.subcore_barrier()
    base0 = (c * 16 + s) * EPT

    def body(k, carry):
        b = base0 + k * CH
        pltpu.sync_copy(src_hbm.at[pl.ds(b, CH)], sidx_v)
        pltpu.async_copy(z_hbm.at[sidx_v], rows_v, sem).wait()
        pltpu.sync_copy(dst_hbm.at[pl.ds(b, CH)], didx_v)
        pltpu.sync_copy(rows_v, acc_sh.at[didx_v], add=True)
        return carry

    lax.fori_loop(0, KCH, body, 0)
    plsc.subcore_barrier()
    pltpu.sync_copy(acc_sh.at[pl.ds(r0, RPT)], out_hbm.at[c, pl.ds(r0, RPT)])


_R = 1024


def _dinv_of(degp):
    return lax.rsqrt(jnp.sum(degp, axis=(0, 2)) + 1.0)[:, None]


def _zscale_body(x_ref, w_ref, degp_ref, z_ref):
    dinv = _dinv_of(degp_ref[...])
    z_ref[...] = jnp.dot(x_ref[...], w_ref[...],
                         preferred_element_type=jnp.float32) * dinv


_zscale = pl.pallas_call(
    _zscale_body,
    grid=(NP // _R,),
    in_specs=[
        pl.BlockSpec((_R, D), lambda i: (i, 0)),
        pl.BlockSpec((D, D), lambda i: (0, 0)),
        pl.BlockSpec((2, _R, DW), lambda i: (0, i, 0)),
    ],
    out_specs=pl.BlockSpec((_R, D), lambda i: (i, 0)),
    out_shape=jax.ShapeDtypeStruct((NP, D), jnp.float32),
)


def _layer2_body(s_ref, z1_ref, degp_ref, b1_ref, w2_ref, z2_ref):
    dinv = _dinv_of(degp_ref[...])
    s = s_ref[...]
    t = (s[0] + s[1] + z1_ref[...]) * dinv + b1_ref[...]
    h = jnp.maximum(t, 0.0)
    z2_ref[...] = jnp.dot(h, w2_ref[...],
                          preferred_element_type=jnp.float32) * dinv


_layer2 = pl.pallas_call(
    _layer2_body,
    grid=(NP // _R,),
    in_specs=[
        pl.BlockSpec((2, _R, D), lambda i: (0, i, 0)),
        pl.BlockSpec((_R, D), lambda i: (i, 0)),
        pl.BlockSpec((2, _R, DW), lambda i: (0, i, 0)),
        pl.BlockSpec((1, D), lambda i: (0, 0)),
        pl.BlockSpec((D, D), lambda i: (0, 0)),
    ],
    out_specs=pl.BlockSpec((_R, D), lambda i: (i, 0)),
    out_shape=jax.ShapeDtypeStruct((NP, D), jnp.float32),
)


def _head_body(s_ref, z2_ref, degp_ref, b2_ref, wl_ref, bl_ref,
               emb_ref, out_ref):
    dinv = _dinv_of(degp_ref[...])
    s = s_ref[...]
    emb = (s[0] + s[1] + z2_ref[...]) * dinv + b2_ref[...]
    emb_ref[...] = emb
    logits = jnp.dot(emb, wl_ref[...],
                     preferred_element_type=jnp.float32) + bl_ref[...]
    mask = lax.broadcasted_iota(jnp.int32, logits.shape, 1) < OUT
    lm = jnp.where(mask, logits, jnp.float32(-1e30))
    m = jnp.max(lm, axis=1, keepdims=True)
    ex = jnp.where(mask, jnp.exp(logits - m), 0.0)
    lse = jnp.log(jnp.sum(ex, axis=1, keepdims=True))
    out_ref[...] = logits - m - lse


_head = pl.pallas_call(
    _head_body,
    grid=(NP // _R,),
    in_specs=[
        pl.BlockSpec((2, _R, D), lambda i: (0, i, 0)),
        pl.BlockSpec((_R, D), lambda i: (i, 0)),
        pl.BlockSpec((2, _R, DW), lambda i: (0, i, 0)),
        pl.BlockSpec((1, D), lambda i: (0, 0)),
        pl.BlockSpec((D, D), lambda i: (0, 0)),
        pl.BlockSpec((1, D), lambda i: (0, 0)),
    ],
    out_specs=[
        pl.BlockSpec((_R, D), lambda i: (i, 0)),
        pl.BlockSpec((_R, D), lambda i: (i, 0)),
    ],
    out_shape=[
        jax.ShapeDtypeStruct((NP, D), jnp.float32),
        jax.ShapeDtypeStruct((NP, D), jnp.float32),
    ],
)


@jax.jit
def kernel(x, edge_index, W1, b1, W2, b2, Wl, bl):
    pad = EP - E
    srcp = jnp.concatenate([edge_index[0], jnp.zeros((pad,), jnp.int32)])
    dstp = jnp.concatenate([edge_index[1], jnp.full((pad,), N, jnp.int32)])
    x_p = jnp.pad(x, ((0, NP - N), (0, 0)))

    e1 = jnp.zeros((CH, DW), jnp.float32).at[:, 0].set(1.0)
    z16 = jnp.zeros((RPT, DW), jnp.float32)
    zrow = jnp.zeros((RPT, D), jnp.float32)

    degp = _deg_kernel(dstp, e1, z16)
    z1 = _zscale(x_p, W1, degp)
    s1 = _mp_kernel(z1, srcp, dstp, zrow)
    z2 = _layer2(s1, z1, degp, b1.reshape(1, D), W2)
    s2 = _mp_kernel(z2, srcp, dstp, zrow)
    wl_p = jnp.pad(Wl, ((0, 0), (0, D - OUT)))
    bl_p = jnp.pad(bl, (0, D - OUT)).reshape(1, D)
    emb, outp = _head(s2, z2, degp, b2.reshape(1, D), wl_p, bl_p)
    return (outp[:N, :OUT], emb[:N])
